# baseline (speedup 1.0000x reference)
import functools

import jax
import jax.numpy as jnp
from jax import lax
from jax.experimental import pallas as pl
from jax.experimental.pallas import tpu as pltpu
from jax.experimental.pallas import tpu_sc as plsc

_NUM_CORES = 2
_NUM_SUBCORES = 16
_LANES = 16
_NBUF = 3


def kernel(x, tkn_table, pos_table):
    B, S = x.shape
    V, D = tkn_table.shape
    NW = _NUM_CORES * _NUM_SUBCORES
    C = S // NW
    H = C // 2
    NCH = B * 2
    assert S % NW == 0 and C % 2 == 0 and D % _LANES == 0

    x = x.astype(jnp.int32)

    mesh = plsc.VectorSubcoreMesh(core_axis_name="c", subcore_axis_name="s")

    @functools.partial(
        pl.kernel,
        mesh=mesh,
        out_type=jax.ShapeDtypeStruct((B, S, D), jnp.float32),
        scratch_types=[
            pltpu.VMEM((B, C), jnp.int32),
            pltpu.VMEM((C, D), jnp.float32),
            pltpu.VMEM((_NBUF, H, D), jnp.float32),
        ]
        + [pltpu.SemaphoreType.DMA] * (2 * _NBUF),
    )
    def emb(x_hbm, tkn_hbm, pos_hbm, out_hbm, idx_v, pos_v, bufs, *sems):
        gsems = sems[:_NBUF]
        wsems = sems[_NBUF:]
        wid = lax.axis_index("s") * _NUM_CORES + lax.axis_index("c")
        s0 = wid * C
        for b in range(B):
            pltpu.sync_copy(x_hbm.at[b, pl.ds(s0, C)], idx_v.at[b])
        pltpu.sync_copy(pos_hbm.at[pl.ds(s0, C)], pos_v)

        gathers = [None] * NCH
        writes = [None] * NCH

        def start_gather(k):
            b, h = divmod(k, 2)
            gathers[k] = pltpu.async_copy(
                tkn_hbm.at[idx_v.at[b, pl.ds(h * H, H)]],
                bufs.at[k % _NBUF],
                gsems[k % _NBUF],
            )

        start_gather(0)
        start_gather(1)
        for k in range(NCH):
            b, h = divmod(k, 2)
            gathers[k].wait()

            def row_body(r, carry, _k=k, _h=h):
                for c in range(D // _LANES):
                    sl = pl.ds(c * _LANES, _LANES)
                    bufs[_k % _NBUF, r, sl] = (
                        bufs[_k % _NBUF, r, sl] + pos_v[_h * H + r, sl]
                    )
                return carry

            lax.fori_loop(0, H, row_body, 0)
            writes[k] = pltpu.async_copy(
                bufs.at[k % _NBUF],
                out_hbm.at[b, pl.ds(s0 + h * H, H)],
                wsems[k % _NBUF],
            )
            if k + 2 < NCH:
                if k >= 1:
                    writes[k - 1].wait()
                start_gather(k + 2)
        for k in range(NCH - _NBUF, NCH):
            writes[k].wait()

    return emb(x, tkn_table, pos_table)
